# baseline (device time: 5105 ns/iter reference)
import jax
import jax.numpy as jnp
from jax.experimental import pallas as pl
from jax.experimental.pallas import tpu as pltpu

K = 8


def kernel(x):
    m, n = x.shape
    bm = m // K

    def body(x_ref, out_ref, buf_ref, sems):
        copies = []
        for k in range(K):
            c = pltpu.make_async_copy(
                x_ref.at[pl.ds(k * bm, bm), :],
                buf_ref.at[pl.ds(k * bm, bm), :],
                sems.at[k],
            )
            c.start()
            copies.append(c)
        for c in copies:
            c.wait()
        out_ref[:, :] = jnp.sum(buf_ref[:, :], axis=0, keepdims=True)

    return pl.pallas_call(
        body,
        out_shape=jax.ShapeDtypeStruct((1, n), jnp.float32),
        in_specs=[pl.BlockSpec(memory_space=pl.MemorySpace.ANY)],
        out_specs=pl.BlockSpec(memory_space=pltpu.VMEM),
        scratch_shapes=[
            pltpu.VMEM((m, n), jnp.float32),
            pltpu.SemaphoreType.DMA((K,)),
        ],
    )(x)
